# Initial kernel scaffold; baseline (speedup 1.0000x reference)
import jax
import jax.numpy as jnp
from jax.experimental import pallas as pl


def kernel(mask_fraction):
    raise NotImplementedError("write your pallas kernel here")


# speedup vs baseline: 3.6648x; 3.6648x over previous
import functools
import math

import jax
import jax.numpy as jnp
from jax.experimental import pallas as pl

_TOP_K = 64
_MASK_THRES = 0.2
_ONE_BITS = 0x3F800000


def _body(x_ref, out_ref, cont_ref, *, s, c, val_iters, idx_iters):
    def bits():
        return jax.lax.bitcast_convert_type(x_ref[...], jnp.int32)

    lo0 = jnp.zeros((1, c), jnp.int32)
    hi0 = jnp.full((1, c), _ONE_BITS, jnp.int32)

    def val_it(_, lh):
        lo, hi = lh
        mid = (lo + hi) >> 1
        cnt = jnp.sum((bits() >= mid).astype(jnp.int32), axis=0, keepdims=True)
        ge = cnt >= _TOP_K
        return (jnp.where(ge, mid, lo), jnp.where(ge, hi, mid))

    lo, _ = jax.lax.fori_loop(0, val_iters, val_it, (lo0, hi0))
    v = lo

    cgt = jnp.sum((bits() > v).astype(jnp.int32), axis=0, keepdims=True)
    r = _TOP_K - cgt

    li0 = jnp.zeros((1, c), jnp.int32)
    hii0 = jnp.full((1, c), s - 1, jnp.int32)

    def idx_it(_, lh):
        li, hii = lh
        mid = (li + hii) >> 1
        rows = jax.lax.broadcasted_iota(jnp.int32, (s, c), 0)
        cnt = jnp.sum(((bits() == v) & (rows <= mid)).astype(jnp.int32),
                      axis=0, keepdims=True)
        ge = cnt >= r
        return (jnp.where(ge, li, mid + 1), jnp.where(ge, mid, hii))

    _, ti = jax.lax.fori_loop(0, idx_iters, idx_it, (li0, hii0))

    rows = jax.lax.broadcasted_iota(jnp.int32, (s, c), 0)
    xb = bits()
    keep = (xb > v) | ((xb == v) & (rows <= ti))
    x = x_ref[...]
    out_ref[...] = jnp.where(keep, x, 0.0)
    cont_ref[...] = (keep & (x >= _MASK_THRES)).astype(jnp.int8)


@jax.jit
def kernel(mask_fraction):
    s, m = mask_fraction.shape
    c = min(128, m)
    val_iters = max(1, math.ceil(math.log2(_ONE_BITS)))
    idx_iters = max(1, math.ceil(math.log2(s)))
    body = functools.partial(_body, s=s, c=c, val_iters=val_iters,
                             idx_iters=idx_iters)
    masked, cont = pl.pallas_call(
        body,
        grid=(m // c,),
        in_specs=[pl.BlockSpec((s, c), lambda j: (0, j))],
        out_specs=[pl.BlockSpec((s, c), lambda j: (0, j)),
                   pl.BlockSpec((s, c), lambda j: (0, j))],
        out_shape=[jax.ShapeDtypeStruct((s, m), jnp.float32),
                   jax.ShapeDtypeStruct((s, m), jnp.int8)],
    )(mask_fraction)
    return masked, cont.astype(jnp.bool_)


# speedup vs baseline: 12.6630x; 3.4553x over previous
import functools

import jax
import jax.numpy as jnp
from jax.experimental import pallas as pl

_TOP_K = 64
_MASK_THRES = 0.2
_ONE_BITS = 0x3F800000
_THRES_BITS = 0x3E4CCCCD
_NGROUP = 64
_SLAB = 2048


def _body(x_ref, out_ref, cont_ref, *, s, c):
    nslab = max(1, s // _SLAB)
    slab_h = s // nslab

    def sbits(k):
        return jax.lax.bitcast_convert_type(
            x_ref[k * slab_h:(k + 1) * slab_h, :], jnp.int32)

    def count_ge(t):
        acc = jnp.zeros((1, c), jnp.int32)
        for k in range(nslab):
            acc = acc + jnp.sum((sbits(k) >= t).astype(jnp.int32), axis=0,
                                keepdims=True)
        return acc

    gh = s // _NGROUP

    def gslab(k):
        return jax.lax.bitcast_convert_type(
            x_ref[k * gh:(k + 1) * gh, :], jnp.int32)

    cmb = gslab(0)
    for k in range(1, _NGROUP):
        cmb = jnp.maximum(cmb, gslab(k))
    hib = jnp.max(cmb, axis=0, keepdims=True) + 1

    cl0 = jnp.zeros((1, c), jnp.int32)
    ch0 = jnp.full((1, c), _ONE_BITS, jnp.int32)

    def cm_it(_, lh):
        lo, hi = lh
        mid = (lo + hi) >> 1
        cnt = jnp.sum((cmb >= mid).astype(jnp.int32), axis=0, keepdims=True)
        ge = cnt >= _TOP_K
        return (jnp.where(ge, mid, lo), jnp.where(ge, hi, mid))

    lob, _ = jax.lax.fori_loop(0, 30, cm_it, (cl0, ch0))

    def v_cond(lh):
        lo, hi = lh
        return jnp.any(hi - lo > 1)

    def v_body(lh):
        lo, hi = lh
        mid = (lo + hi) >> 1
        ge = count_ge(mid) >= _TOP_K
        return (jnp.where(ge, mid, lo), jnp.where(ge, hi, mid))

    v, _ = jax.lax.while_loop(v_cond, v_body, (lob, hib))

    cgt = count_ge(v + 1)
    r = _TOP_K - cgt

    def rows_iota(k):
        return (jax.lax.broadcasted_iota(jnp.int32, (slab_h, c), 0)
                + k * slab_h)

    first_eq = jnp.full((1, c), s, jnp.int32)
    last_eq = jnp.full((1, c), -1, jnp.int32)
    for k in range(nslab):
        eq = sbits(k) == v
        rk = rows_iota(k)
        first_eq = jnp.minimum(
            first_eq, jnp.min(jnp.where(eq, rk, s), axis=0, keepdims=True))
        last_eq = jnp.maximum(
            last_eq, jnp.max(jnp.where(eq, rk, -1), axis=0, keepdims=True))

    tie = r > 1
    li0 = jnp.where(tie, first_eq + 1, 0)
    hi0 = jnp.where(tie, last_eq, 0)

    def t_cond(lh):
        li, hii = lh
        return jnp.any(hii > li)

    def t_body(lh):
        li, hii = lh
        mid = (li + hii) >> 1
        acc = jnp.zeros((1, c), jnp.int32)
        for k in range(nslab):
            pref = (sbits(k) == v) & (rows_iota(k) <= mid)
            acc = acc + jnp.sum(pref.astype(jnp.int32), axis=0, keepdims=True)
        ge = acc >= r
        return (jnp.where(ge, li, mid + 1), jnp.where(ge, mid, hii))

    _, tib = jax.lax.while_loop(t_cond, t_body, (li0, hi0))
    ti = jnp.where(tie, tib, first_eq)

    for k in range(nslab):
        xb = sbits(k)
        keep = (xb > v) | ((xb == v) & (rows_iota(k) <= ti))
        sl = slice(k * slab_h, (k + 1) * slab_h)
        out_ref[sl, :] = jax.lax.bitcast_convert_type(
            jnp.where(keep, xb, 0), jnp.float32)
        cont_ref[sl, :] = (keep & (xb >= _THRES_BITS)).astype(jnp.int8)


@jax.jit
def kernel(mask_fraction):
    s, m = mask_fraction.shape
    c = min(128, m)
    body = functools.partial(_body, s=s, c=c)
    masked, cont = pl.pallas_call(
        body,
        grid=(m // c,),
        in_specs=[pl.BlockSpec((s, c), lambda j: (0, j))],
        out_specs=[pl.BlockSpec((s, c), lambda j: (0, j)),
                   pl.BlockSpec((s, c), lambda j: (0, j))],
        out_shape=[jax.ShapeDtypeStruct((s, m), jnp.float32),
                   jax.ShapeDtypeStruct((s, m), jnp.int8)],
    )(mask_fraction)
    return masked, cont.astype(jnp.bool_)


# speedup vs baseline: 14.3700x; 1.1348x over previous
import functools

import jax
import jax.numpy as jnp
from jax.experimental import pallas as pl

_TOP_K = 64
_MASK_THRES = 0.2
_ONE_BITS = 0x3F800000
_THRES_BITS = 0x3E4CCCCD
_NGROUP = 64
_SLAB = 2048


def _body(x_ref, out_ref, cont_ref, *, s, c):
    nslab = max(1, s // _SLAB)
    slab_h = s // nslab

    def sbits(k):
        return jax.lax.bitcast_convert_type(
            x_ref[k * slab_h:(k + 1) * slab_h, :], jnp.int32)

    def count_ge(t):
        acc = jnp.zeros((1, c), jnp.int32)
        for k in range(nslab):
            acc = acc + jnp.sum((sbits(k) >= t).astype(jnp.int32), axis=0,
                                keepdims=True)
        return acc

    gh = s // _NGROUP

    def gslab(k):
        return jax.lax.bitcast_convert_type(
            x_ref[k * gh:(k + 1) * gh, :], jnp.int32)

    cmb = gslab(0)
    for k in range(1, _NGROUP):
        cmb = jnp.maximum(cmb, gslab(k))
    hib = jnp.max(cmb, axis=0, keepdims=True) + 1

    cl0 = jnp.zeros((1, c), jnp.int32)
    ch0 = jnp.full((1, c), _ONE_BITS, jnp.int32)

    def cm_it(_, lh):
        lo, hi = lh
        mid = (lo + hi) >> 1
        cnt = jnp.sum((cmb >= mid).astype(jnp.int32), axis=0, keepdims=True)
        ge = cnt >= _TOP_K
        return (jnp.where(ge, mid, lo), jnp.where(ge, hi, mid))

    lob, _ = jax.lax.fori_loop(0, 30, cm_it, (cl0, ch0))

    def v_cond(lhc):
        lo, hi, _ = lhc
        return jnp.any(hi - lo > 1)

    def v_body(lhc):
        lo, hi, cnt_hi = lhc
        mid = (lo + hi) >> 1
        cnt = count_ge(mid)
        ge = cnt >= _TOP_K
        return (jnp.where(ge, mid, lo), jnp.where(ge, hi, mid),
                jnp.where(ge, cnt_hi, cnt))

    v, _, cgt = jax.lax.while_loop(
        v_cond, v_body, (lob, hib, jnp.zeros((1, c), jnp.int32)))

    r = _TOP_K - cgt

    def rows_iota(k):
        return (jax.lax.broadcasted_iota(jnp.int32, (slab_h, c), 0)
                + k * slab_h)

    first_eq = jnp.full((1, c), s, jnp.int32)
    for k in range(nslab):
        eq = sbits(k) == v
        first_eq = jnp.minimum(
            first_eq,
            jnp.min(jnp.where(eq, rows_iota(k), s), axis=0, keepdims=True))

    tie = r > 1
    li0 = jnp.where(tie, first_eq + 1, 0)
    hi0 = jnp.where(tie, s - 1, 0)

    def t_cond(lh):
        li, hii = lh
        return jnp.any(hii > li)

    def t_body(lh):
        li, hii = lh
        mid = (li + hii) >> 1
        acc = jnp.zeros((1, c), jnp.int32)
        for k in range(nslab):
            pref = (sbits(k) == v) & (rows_iota(k) <= mid)
            acc = acc + jnp.sum(pref.astype(jnp.int32), axis=0, keepdims=True)
        ge = acc >= r
        return (jnp.where(ge, li, mid + 1), jnp.where(ge, mid, hii))

    _, tib = jax.lax.while_loop(t_cond, t_body, (li0, hi0))
    ti = jnp.where(tie, tib, first_eq)

    for k in range(nslab):
        xb = sbits(k)
        keep = xb >= (v + (rows_iota(k) > ti).astype(jnp.int32))
        sl = slice(k * slab_h, (k + 1) * slab_h)
        out_ref[sl, :] = jax.lax.bitcast_convert_type(
            jnp.where(keep, xb, 0), jnp.float32)
        cont_ref[sl, :] = (keep & (xb >= _THRES_BITS)).astype(jnp.int8)


@jax.jit
def kernel(mask_fraction):
    s, m = mask_fraction.shape
    c = min(128, m)
    body = functools.partial(_body, s=s, c=c)
    masked, cont = pl.pallas_call(
        body,
        grid=(m // c,),
        in_specs=[pl.BlockSpec((s, c), lambda j: (0, j))],
        out_specs=[pl.BlockSpec((s, c), lambda j: (0, j)),
                   pl.BlockSpec((s, c), lambda j: (0, j))],
        out_shape=[jax.ShapeDtypeStruct((s, m), jnp.float32),
                   jax.ShapeDtypeStruct((s, m), jnp.int8)],
    )(mask_fraction)
    return masked, cont.astype(jnp.bool_)


# speedup vs baseline: 15.9329x; 1.1088x over previous
import functools

import jax
import jax.numpy as jnp
from jax.experimental import pallas as pl

_TOP_K = 64
_MASK_THRES = 0.2
_ONE_BITS = 0x3F800000
_THRES_BITS = 0x3E4CCCCD
_NGROUP = 64
_SLAB = 2048


def _body(x_ref, out_ref, cont_ref, *, s, c):
    nslab = max(1, s // _SLAB)
    slab_h = s // nslab

    def sbits(k):
        return jax.lax.bitcast_convert_type(
            x_ref[k * slab_h:(k + 1) * slab_h, :], jnp.int32)

    ones_row = jnp.ones((1, slab_h), jnp.bfloat16)

    def count_ge(t):
        acc = jnp.zeros((1, c), jnp.float32)
        for k in range(nslab):
            ind = (sbits(k) >= t).astype(jnp.bfloat16)
            acc = acc + jax.lax.dot_general(
                ones_row, ind, (((1,), (0,)), ((), ())),
                preferred_element_type=jnp.float32)
        return acc.astype(jnp.int32)

    gh = s // _NGROUP

    def gslab(k):
        return jax.lax.bitcast_convert_type(
            x_ref[k * gh:(k + 1) * gh, :], jnp.int32)

    cmb = gslab(0)
    for k in range(1, _NGROUP):
        cmb = jnp.maximum(cmb, gslab(k))
    hib = jnp.max(cmb, axis=0, keepdims=True) + 1

    cl0 = jnp.zeros((1, c), jnp.int32)
    ch0 = jnp.full((1, c), _ONE_BITS, jnp.int32)

    def cm_it(_, lh):
        lo, hi = lh
        mid = (lo + hi) >> 1
        cnt = jnp.sum((cmb >= mid).astype(jnp.int32), axis=0, keepdims=True)
        ge = cnt >= _TOP_K
        return (jnp.where(ge, mid, lo), jnp.where(ge, hi, mid))

    lob, _ = jax.lax.fori_loop(0, 30, cm_it, (cl0, ch0))

    def v_cond(lhc):
        lo, hi, _ = lhc
        return jnp.any(hi - lo > 1)

    def v_body(lhc):
        lo, hi, cnt_hi = lhc
        mid = (lo + hi) >> 1
        cnt = count_ge(mid)
        ge = cnt >= _TOP_K
        return (jnp.where(ge, mid, lo), jnp.where(ge, hi, mid),
                jnp.where(ge, cnt_hi, cnt))

    v, _, cgt = jax.lax.while_loop(
        v_cond, v_body, (lob, hib, jnp.zeros((1, c), jnp.int32)))

    r = _TOP_K - cgt

    def rows_iota(k):
        return (jax.lax.broadcasted_iota(jnp.int32, (slab_h, c), 0)
                + k * slab_h)

    first_eq = jnp.full((1, c), s, jnp.int32)
    for k in range(nslab):
        eq = sbits(k) == v
        first_eq = jnp.minimum(
            first_eq,
            jnp.min(jnp.where(eq, rows_iota(k), s), axis=0, keepdims=True))

    tie = r > 1
    li0 = jnp.where(tie, first_eq + 1, 0)
    hi0 = jnp.where(tie, s - 1, 0)

    def t_cond(lh):
        li, hii = lh
        return jnp.any(hii > li)

    def t_body(lh):
        li, hii = lh
        mid = (li + hii) >> 1
        acc = jnp.zeros((1, c), jnp.int32)
        for k in range(nslab):
            pref = (sbits(k) == v) & (rows_iota(k) <= mid)
            acc = acc + jnp.sum(pref.astype(jnp.int32), axis=0, keepdims=True)
        ge = acc >= r
        return (jnp.where(ge, li, mid + 1), jnp.where(ge, mid, hii))

    _, tib = jax.lax.while_loop(t_cond, t_body, (li0, hi0))
    ti = jnp.where(tie, tib, first_eq)

    for k in range(nslab):
        xb = sbits(k)
        keep = xb >= (v + (rows_iota(k) > ti).astype(jnp.int32))
        sl = slice(k * slab_h, (k + 1) * slab_h)
        out_ref[sl, :] = jax.lax.bitcast_convert_type(
            jnp.where(keep, xb, 0), jnp.float32)
        cont_ref[sl, :] = (keep & (xb >= _THRES_BITS)).astype(jnp.int8)


@jax.jit
def kernel(mask_fraction):
    s, m = mask_fraction.shape
    c = min(128, m)
    body = functools.partial(_body, s=s, c=c)
    masked, cont = pl.pallas_call(
        body,
        grid=(m // c,),
        in_specs=[pl.BlockSpec((s, c), lambda j: (0, j))],
        out_specs=[pl.BlockSpec((s, c), lambda j: (0, j)),
                   pl.BlockSpec((s, c), lambda j: (0, j))],
        out_shape=[jax.ShapeDtypeStruct((s, m), jnp.float32),
                   jax.ShapeDtypeStruct((s, m), jnp.int8)],
    )(mask_fraction)
    return masked, cont.astype(jnp.bool_)
